# speedup vs baseline: 173.0843x; 173.0843x over previous
import functools

import jax
import jax.numpy as jnp
from jax import lax
from jax.experimental import pallas as pl
from jax.experimental.pallas import tpu as pltpu
from jax.experimental.pallas import tpu_sc as plsc

T = 26
B = 4096
L = 20
V = 100000
D = 128

NC = 2
NS = 16
NW = NC * NS

BAGS_W = B // NW
IDX_W = BAGS_W * L
NB = 16
RPC = NB * L
NCHUNK = BAGS_W // NB
DMA_ROWS = 80
DPC = RPC // DMA_ROWS
IDX_ROWS_W = IDX_W // DMA_ROWS
IDX_ROWS_T = B * L // DMA_ROWS


def _dlrm_pool_sc(idx2d, tables2d):
    mesh = plsc.VectorSubcoreMesh(core_axis_name="c", subcore_axis_name="s")

    @functools.partial(
        pl.kernel,
        mesh=mesh,
        out_type=jax.ShapeDtypeStruct((T * B, D), jnp.float32),
        scratch_types=[
            pltpu.VMEM((IDX_ROWS_W, DMA_ROWS), jnp.int32),
            pltpu.VMEM((RPC, D), jnp.float32),
            pltpu.VMEM((NB, D), jnp.float32),
            pltpu.SemaphoreType.DMA,
        ],
    )
    def body(idx_hbm, tbl_hbm, out_hbm, idx_v, rows_v, pool_v, gsem):
        wid = lax.axis_index("s") * NC + lax.axis_index("c")

        def table_body(t, carry):
            pltpu.sync_copy(
                idx_hbm.at[pl.ds(t * IDX_ROWS_T + wid * IDX_ROWS_W, IDX_ROWS_W), :],
                idx_v,
            )

            def chunk_body(c, carry2):
                copies = [
                    pltpu.async_copy(
                        tbl_hbm.at[idx_v.at[c * DPC + q]],
                        rows_v.at[pl.ds(q * DMA_ROWS, DMA_ROWS), :],
                        gsem,
                    )
                    for q in range(DPC)
                ]
                for cp in copies:
                    cp.wait()

                def bag_body(b, carry3):
                    rbase = b * L
                    for col in range(D // 16):
                        sl = pl.ds(col * 16, 16)
                        acc = rows_v[rbase, sl]
                        for j in range(1, L):
                            acc = acc + rows_v[rbase + j, sl]
                        pool_v[b, sl] = acc
                    return carry3

                lax.fori_loop(0, NB, bag_body, 0)
                pltpu.sync_copy(
                    pool_v,
                    out_hbm.at[pl.ds(t * B + wid * BAGS_W + c * NB, NB), :],
                )
                return carry2

            lax.fori_loop(0, NCHUNK, chunk_body, 0)
            return carry

        lax.fori_loop(0, T, table_body, 0)

    return body(idx2d, tables2d)


def kernel(lS_i, lS_o, emb_tables):
    del lS_o
    flat_idx = lS_i + (jnp.arange(T, dtype=jnp.int32) * V)[:, None]
    idx2d = flat_idx.reshape(T * B * L // DMA_ROWS, DMA_ROWS)
    tables2d = emb_tables.reshape(T * V, D)
    out = _dlrm_pool_sc(idx2d, tables2d)
    return out.reshape(T, B, D)
